# baseline (device time: 46654 ns/iter reference)
import jax
import jax.numpy as jnp
from jax import lax
from jax.experimental import pallas as pl
from jax.experimental.pallas import tpu as pltpu

N_DEV = 32
B = 512
D = 256
R = B // N_DEV
N_PHASES = 5


def kernel(x, Win0, Wout0, Win1, Wout1, Win2, Wout2):
    def body(x_ref, win0_ref, wout0_ref, win1_ref, wout1_ref, win2_ref,
             wout2_ref, out_ref, accum_ref, rs_ref, xbuf_ref,
             send_sems, recv_sems):
        me = lax.axis_index("i")

        barrier_sem = pltpu.get_barrier_semaphore()

        def _signal(k, c):
            tgt = lax.rem(me + k, N_DEV)
            pl.semaphore_signal(
                barrier_sem, inc=1,
                device_id=(tgt,), device_id_type=pl.DeviceIdType.MESH,
            )
            return c
        lax.fori_loop(1, N_DEV, _signal, 0)
        pl.semaphore_wait(barrier_sem, N_DEV - 1)

        def rs_send_desc(k, phase):
            tgt = lax.rem(me + k, N_DEV)
            return pltpu.make_async_remote_copy(
                src_ref=accum_ref.at[pl.ds(tgt * R, R), :],
                dst_ref=rs_ref.at[pl.ds(me * R, R), :],
                send_sem=send_sems.at[phase],
                recv_sem=recv_sems.at[phase],
                device_id=(tgt,),
                device_id_type=pl.DeviceIdType.MESH,
            )

        def rs_recv_desc(k, phase):
            src = lax.rem(me + k, N_DEV)
            return pltpu.make_async_remote_copy(
                src_ref=accum_ref.at[pl.ds(0, R), :],
                dst_ref=rs_ref.at[pl.ds(src * R, R), :],
                send_sem=send_sems.at[phase],
                recv_sem=recv_sems.at[phase],
                device_id=(src,),
                device_id_type=pl.DeviceIdType.MESH,
            )

        row_i = lax.broadcasted_iota(jnp.int32, (R, B), 0)
        col_c = lax.broadcasted_iota(jnp.int32, (R, B), 1)
        psel = (lax.rem(col_c, R) == row_i).astype(jnp.bfloat16)

        def ag_send_desc(k, phase):
            tgt = lax.rem(me + k, N_DEV)
            return pltpu.make_async_remote_copy(
                src_ref=xbuf_ref.at[pl.ds(me * R, R), :],
                dst_ref=xbuf_ref.at[pl.ds(me * R, R), :],
                send_sem=send_sems.at[phase],
                recv_sem=recv_sems.at[phase],
                device_id=(tgt,),
                device_id_type=pl.DeviceIdType.MESH,
            )

        def ag_recv_desc(k, phase):
            src = lax.rem(me + k, N_DEV)
            return pltpu.make_async_remote_copy(
                src_ref=xbuf_ref.at[pl.ds(0, R), :],
                dst_ref=xbuf_ref.at[pl.ds(src * R, R), :],
                send_sem=send_sems.at[phase],
                recv_sem=recv_sems.at[phase],
                device_id=(src,),
                device_id_type=pl.DeviceIdType.MESH,
            )

        def rs_phase(phase):
            for k in range(1, N_DEV):
                rs_send_desc(k, phase).start()
            rs_ref[pl.ds(me * R, R), :] = accum_ref[pl.ds(me * R, R), :]
            for k in range(1, N_DEV):
                rs_send_desc(k, phase).wait_send()
                rs_recv_desc(k, phase).wait_recv()
            return jnp.dot(psel, rs_ref[...],
                           preferred_element_type=jnp.float32)

        def ag_phase(phase, y):
            xbuf_ref[pl.ds(me * R, R), :] = y.astype(jnp.bfloat16)
            for k in range(1, N_DEV):
                ag_send_desc(k, phase).start()
            for k in range(1, N_DEV):
                ag_send_desc(k, phase).wait_send()
                ag_recv_desc(k, phase).wait_recv()

        def layer(xv, win_ref, wout_ref):
            h = jnp.dot(xv.astype(jnp.bfloat16),
                        win_ref[...].astype(jnp.bfloat16),
                        preferred_element_type=jnp.float32)
            h = jnp.maximum(h, 0.0)
            p = jnp.dot(h.astype(jnp.bfloat16),
                        wout_ref[...].astype(jnp.bfloat16),
                        preferred_element_type=jnp.float32)
            return p.astype(jnp.bfloat16)

        accum_ref[...] = layer(x_ref[...], win0_ref, wout0_ref)
        y0 = rs_phase(0)
        ag_phase(1, y0)

        accum_ref[...] = layer(xbuf_ref[...], win1_ref, wout1_ref)
        y1 = rs_phase(2)
        ag_phase(3, y1)

        accum_ref[...] = layer(xbuf_ref[...], win2_ref, wout2_ref)
        out_ref[...] = rs_phase(4)

    return pl.pallas_call(
        body,
        out_shape=jax.ShapeDtypeStruct((R, D), jnp.float32),
        in_specs=[pl.BlockSpec(memory_space=pltpu.VMEM)] * 7,
        out_specs=pl.BlockSpec(memory_space=pltpu.VMEM),
        scratch_shapes=[
            pltpu.VMEM((B, D), jnp.bfloat16),
            pltpu.VMEM((B, D), jnp.bfloat16),
            pltpu.VMEM((B, D), jnp.bfloat16),
            pltpu.SemaphoreType.DMA((N_PHASES,)),
            pltpu.SemaphoreType.DMA((N_PHASES,)),
        ],
        compiler_params=pltpu.CompilerParams(collective_id=0),
    )(x, Win0, Wout0, Win1, Wout1, Win2, Wout2)


# device time: 44236 ns/iter; 1.0547x vs baseline; 1.0547x over previous
import jax
import jax.numpy as jnp
from jax import lax
from jax.experimental import pallas as pl
from jax.experimental.pallas import tpu as pltpu

N_DEV = 32
B = 512
D = 256
R = B // N_DEV
C = 2
G = N_DEV // C
BC = B // C
N_PHASES = 5


def kernel(x, Win0, Wout0, Win1, Wout1, Win2, Wout2):
    def body(x_ref, win0_ref, wout0_ref, win1_ref, wout1_ref, win2_ref,
             wout2_ref, out_ref, accum_ref, rs_ref, xbuf_ref,
             send_sems, recv_sems, ag_sems):
        me = lax.axis_index("i")
        c_me = me // G

        barrier_sem = pltpu.get_barrier_semaphore()

        def _signal(k, c):
            tgt = lax.rem(me + k, N_DEV)
            pl.semaphore_signal(
                barrier_sem, inc=1,
                device_id=(tgt,), device_id_type=pl.DeviceIdType.MESH,
            )
            return c
        lax.fori_loop(1, N_DEV, _signal, 0)

        row_i = lax.broadcasted_iota(jnp.int32, (R, B), 0)
        col_c = lax.broadcasted_iota(jnp.int32, (R, B), 1)
        psel = (lax.rem(col_c, R) == row_i).astype(jnp.bfloat16)

        def rs_send_desc(t, phase):
            return pltpu.make_async_remote_copy(
                src_ref=accum_ref.at[pl.ds(t * R, R), :],
                dst_ref=rs_ref.at[pl.ds(me * R, R), :],
                send_sem=send_sems.at[phase],
                recv_sem=recv_sems.at[phase],
                device_id=(t,),
                device_id_type=pl.DeviceIdType.MESH,
            )

        def rs_recv_desc(s, phase):
            return pltpu.make_async_remote_copy(
                src_ref=accum_ref.at[pl.ds(0, R), :],
                dst_ref=rs_ref.at[pl.ds(s * R, R), :],
                send_sem=send_sems.at[phase],
                recv_sem=recv_sems.at[phase],
                device_id=(s,),
                device_id_type=pl.DeviceIdType.MESH,
            )

        def ag_send_desc(t, phase, sem_idx):
            return pltpu.make_async_remote_copy(
                src_ref=xbuf_ref.at[pl.ds(me * R, R), :],
                dst_ref=xbuf_ref.at[pl.ds(me * R, R), :],
                send_sem=send_sems.at[phase],
                recv_sem=ag_sems.at[sem_idx],
                device_id=(t,),
                device_id_type=pl.DeviceIdType.MESH,
            )

        def ag_recv_desc(s, sem_idx):
            return pltpu.make_async_remote_copy(
                src_ref=xbuf_ref.at[pl.ds(0, R), :],
                dst_ref=xbuf_ref.at[pl.ds(s * R, R), :],
                send_sem=send_sems.at[0],
                recv_sem=ag_sems.at[sem_idx],
                device_id=(s,),
                device_id_type=pl.DeviceIdType.MESH,
            )

        def rs_sends_chunk(c, phase):
            for k in range(G):
                t = c * G + lax.rem(me + k, G)

                @pl.when(t != me)
                def _():
                    rs_send_desc(t, phase).start()

            @pl.when(c_me == c)
            def _():
                rs_ref[pl.ds(me * R, R), :] = accum_ref[pl.ds(me * R, R), :]

        def rs_finish(phase):
            for k in range(1, N_DEV):
                peer = lax.rem(me + k, N_DEV)
                rs_send_desc(peer, phase).wait_send()
                rs_recv_desc(peer, phase).wait_recv()
            return jnp.dot(psel, rs_ref[...],
                           preferred_element_type=jnp.float32)

        def ag_phase(slot, phase, y):
            xbuf_ref[pl.ds(me * R, R), :] = y.astype(jnp.bfloat16)
            ks_far_first = sorted(range(1, N_DEV),
                                  key=lambda k: (-min(k, N_DEV - k), k))
            for cc in range(C):
                @pl.when(c_me == cc)
                def _():
                    for k in ks_far_first:
                        t = lax.rem(me + k, N_DEV)
                        ag_send_desc(t, phase, slot * C + cc).start()
            for cc in range(C):
                @pl.when(c_me == cc)
                def _():
                    for k in range(1, N_DEV):
                        t = lax.rem(me + k, N_DEV)
                        ag_send_desc(t, phase, slot * C + cc).wait_send()

        def ag_wait_chunk(slot, c):
            for k in range(G):
                s = c * G + lax.rem(me + k, G)

                @pl.when(s != me)
                def _():
                    ag_recv_desc(s, slot * C + c).wait_recv()

        def compute_chunk(c, xv_bf, w_in, w_out):
            h = jnp.dot(xv_bf, w_in, preferred_element_type=jnp.float32)
            h = jnp.maximum(h, 0.0)
            p = jnp.dot(h.astype(jnp.bfloat16), w_out,
                        preferred_element_type=jnp.float32)
            accum_ref[pl.ds(c * BC, BC), :] = p.astype(jnp.bfloat16)

        x_bf = x_ref[...]
        w_in = win0_ref[...]
        w_out = wout0_ref[...]
        for c in range(C):
            compute_chunk(c, x_bf[c * BC:(c + 1) * BC, :], w_in, w_out)
            if c == 0:
                pl.semaphore_wait(barrier_sem, N_DEV - 1)
            rs_sends_chunk(c, 0)
        y = rs_finish(0)
        ag_phase(0, 1, y)

        for l, (wi_ref, wo_ref) in ((1, (win1_ref, wout1_ref)),
                                    (2, (win2_ref, wout2_ref))):
            w_in = wi_ref[...]
            w_out = wo_ref[...]
            for c in range(C):
                ag_wait_chunk(l - 1, c)
                compute_chunk(c, xbuf_ref[pl.ds(c * BC, BC), :], w_in, w_out)
                rs_sends_chunk(c, 2 * l)
            y = rs_finish(2 * l)
            if l < 2:
                ag_phase(l, 2 * l + 1, y)

        out_ref[...] = y

    x = x.astype(jnp.bfloat16)
    Win0 = Win0.astype(jnp.bfloat16)
    Wout0 = Wout0.astype(jnp.bfloat16)
    Win1 = Win1.astype(jnp.bfloat16)
    Wout1 = Wout1.astype(jnp.bfloat16)
    Win2 = Win2.astype(jnp.bfloat16)
    Wout2 = Wout2.astype(jnp.bfloat16)
    return pl.pallas_call(
        body,
        out_shape=jax.ShapeDtypeStruct((R, D), jnp.float32),
        in_specs=[pl.BlockSpec(memory_space=pltpu.VMEM)] * 7,
        out_specs=pl.BlockSpec(memory_space=pltpu.VMEM),
        scratch_shapes=[
            pltpu.VMEM((B, D), jnp.bfloat16),
            pltpu.VMEM((B, D), jnp.bfloat16),
            pltpu.VMEM((B, D), jnp.bfloat16),
            pltpu.SemaphoreType.DMA((N_PHASES,)),
            pltpu.SemaphoreType.DMA((N_PHASES,)),
            pltpu.SemaphoreType.DMA((2 * C,)),
        ],
        compiler_params=pltpu.CompilerParams(collective_id=0),
    )(x, Win0, Wout0, Win1, Wout1, Win2, Wout2)
